# baseline (speedup 1.0000x reference)
import jax
import jax.numpy as jnp
from jax import lax
from jax.experimental import pallas as pl
from jax.experimental.pallas import tpu as pltpu
from jax.experimental.pallas import tpu_sc as plsc

N = 50000
E = 800000
H2 = 64
ROW = 128
NROWS = 6400
EPAD = NROWS * ROW
RROWS = E // ROW
NC, NS = 2, 16
NW = NC * NS
HALF = N // NC
ACC_ROWS = 25128
BIGROW = 25000
CROWS = EPAD // 2

RPW = 196
KBA = 7
PW = 16
PSTRIPE = 3128

HROWS = NROWS // 2
CRPT = HROWS // NS
RREAL = 3050
CB = 48
ZCH = 33
DCH = 33
TSTRIDE = 1568
ZSTRIDE = 1576

_SC_PARAMS = pltpu.CompilerParams(use_tc_tiling_on_sc=False,
                                  needs_layout_passes=False)


def _rsqrt_sc(d2):
    bits = plsc.bitcast(d2, jnp.int32)
    y = plsc.bitcast(0x5F3759DF - (bits >> 1), jnp.float32)
    for _ in range(3):
        y = y * (1.5 - (0.5 * d2 * y) * y)
    return y


def _gather_body(pos_hbm, src_hbm, dst_hbm, posU_hbm,
                 sidx, didx, sbuf, dbuf, planes, pos_sh, sem):
    c = lax.axis_index("c")
    s = lax.axis_index("s")
    wid = s * NC + c
    iota = lax.iota(jnp.int32, 16)

    soff = jnp.minimum(s * PSTRIPE, N - PSTRIPE)
    pltpu.sync_copy(pos_hbm.at[pl.ds(soff, PSTRIPE)],
                    pos_sh.at[pl.ds(soff, PSTRIPE)])
    plsc.subcore_barrier()

    start = jnp.minimum(wid * RPW, RROWS - RPW)

    def block(b, carry):
        r0 = start + b * KBA
        pltpu.sync_copy(src_hbm.at[pl.ds(r0 * ROW, KBA * ROW)], sidx)
        pltpu.sync_copy(dst_hbm.at[pl.ds(r0 * ROW, KBA * ROW)], didx)
        descs = []
        for j in range(KBA):
            descs.append(pltpu.async_copy(
                pos_sh.at[sidx.at[pl.ds(j * ROW, ROW)]],
                sbuf.at[pl.ds(j * ROW, ROW)], sem))
            descs.append(pltpu.async_copy(
                pos_sh.at[didx.at[pl.ds(j * ROW, ROW)]],
                dbuf.at[pl.ds(j * ROW, ROW)], sem))
        for d in descs:
            d.wait()

        def grp(g, carry2):
            ridx = g * 16 + iota
            grow = 3 * (g // 8)
            lo = (g % 8) * 16
            comp = []
            for cc in range(3):
                cidx = jnp.full((16,), cc, jnp.int32)
                xs = plsc.load_gather(sbuf, [ridx, cidx])
                xd = plsc.load_gather(dbuf, [ridx, cidx])
                comp.append(xd - xs)
            d2 = comp[0] * comp[0] + comp[1] * comp[1] + comp[2] * comp[2]
            r = _rsqrt_sc(d2)
            f = 1.0 / (d2 * r + 1e-6)
            for cc in range(3):
                planes[grow + cc, pl.ds(lo, 16)] = comp[cc] * f
            return carry2
        lax.fori_loop(0, KBA * 8, grp, 0)
        pltpu.sync_copy(planes, posU_hbm.at[pl.ds(3 * r0, 3 * KBA)])
        return carry

    lax.fori_loop(0, RPW // KBA, block, 0)


@jax.jit
def _gather_call(pos16, srcv, dstv):
    f = pl.kernel(
        _gather_body,
        out_type=jax.ShapeDtypeStruct((3 * NROWS, ROW), jnp.float32),
        mesh=plsc.VectorSubcoreMesh(core_axis_name="c", subcore_axis_name="s"),
        scratch_types=[
            pltpu.VMEM((KBA * ROW,), jnp.int32),
            pltpu.VMEM((KBA * ROW,), jnp.int32),
            pltpu.VMEM((KBA * ROW, PW), jnp.float32),
            pltpu.VMEM((KBA * ROW, PW), jnp.float32),
            pltpu.VMEM((3 * KBA, ROW), jnp.float32),
            pltpu.VMEM_SHARED((N, PW), jnp.float32),
            pltpu.SemaphoreType.DMA,
        ],
        compiler_params=_SC_PARAMS,
    )
    return f(pos16, srcv, dstv)


BE = 4096


def _mlp_body(w1_ref, b1_ref, gam2_ref, bet2_ref, w2d_ref, b22_ref,
              ul_ref, ur_ref, o_ref):
    w1 = w1_ref[...]
    b1 = b1_ref[...]
    gam2 = gam2_ref[...]
    bet2 = bet2_ref[...]
    w2d = w2d_ref[...]
    b22 = b22_ref[...]

    def mean64(v):
        return jnp.mean(v, axis=1, keepdims=True)

    x0, x1, x2 = w1[0:1], w1[1:2], w1[2:3]
    z1 = jnp.zeros((1, 1), jnp.float32)
    muW = jnp.concatenate(
        [mean64(x0), mean64(x1), mean64(x2),
         z1, z1, z1, z1, z1, z1, mean64(b1)], axis=0)
    qW = jnp.concatenate(
        [2 * mean64(b1 * x0), 2 * mean64(b1 * x1), 2 * mean64(b1 * x2),
         mean64(x0 * x0), mean64(x1 * x1), mean64(x2 * x2),
         2 * mean64(x0 * x1), 2 * mean64(x0 * x2), 2 * mean64(x1 * x2),
         mean64(b1 * b1)], axis=0)
    Wmq = jnp.concatenate([jnp.broadcast_to(muW, (10, H2)),
                           jnp.broadcast_to(qW, (10, H2))], axis=1)
    W1a = jnp.concatenate([w1, jnp.zeros((6, H2), jnp.float32), b1], axis=0)
    ones128 = jnp.ones((1, ROW), jnp.float32)

    def half(u_ref):
        hs, ms = [], []
        for g in range(BE // ROW):
            ug = u_ref[3 * g:3 * g + 3, :]
            ux, uy, uz = ug[0:1], ug[1:2], ug[2:3]
            F = jnp.concatenate(
                [ug, ux * ux, uy * uy, uz * uz,
                 ux * uy, ux * uz, uy * uz, ones128], axis=0)
            hs.append(lax.dot_general(F, W1a, (((0,), (0,)), ((), ())),
                                      preferred_element_type=jnp.float32))
            ms.append(lax.dot_general(F, Wmq, (((0,), (0,)), ((), ())),
                                      preferred_element_type=jnp.float32))
        return jnp.concatenate(hs, axis=0), jnp.concatenate(ms, axis=0)

    HL, ML = half(ul_ref)
    HR, MR = half(ur_ref)
    H = jnp.concatenate([HL, HR], axis=1)
    MU = jnp.concatenate([ML[:, :H2], MR[:, :H2]], axis=1)
    Q = jnp.concatenate([ML[:, H2:], MR[:, H2:]], axis=1)
    var = Q - MU * MU
    hn = (H - MU) * lax.rsqrt(var + 1e-5) * gam2 + bet2
    g = 0.5 * hn * (1.0 + lax.erf(hn * 0.7071067811865476))
    o_ref[...] = lax.dot_general(g, w2d, (((1,), (0,)), ((), ())),
                                 preferred_element_type=jnp.float32) + b22


@jax.jit
def _mlp_call(W1, b1, gamma, beta, W2, b2, posU):
    grid = (CROWS // BE,)
    full = lambda i: (0, 0)
    return pl.pallas_call(
        _mlp_body,
        grid=grid,
        in_specs=[
            pl.BlockSpec((3, H2), full),
            pl.BlockSpec((1, H2), full),
            pl.BlockSpec((1, 2 * H2), full),
            pl.BlockSpec((1, 2 * H2), full),
            pl.BlockSpec((2 * H2, 2 * H2), full),
            pl.BlockSpec((1, 2 * H2), full),
            pl.BlockSpec((3 * BE // ROW, ROW), lambda i: (i, 0)),
            pl.BlockSpec((3 * BE // ROW, ROW),
                         lambda i: (i + CROWS // BE, 0)),
        ],
        out_specs=pl.BlockSpec((BE, 2 * H2), lambda i: (i, 0)),
        out_shape=jax.ShapeDtypeStruct((CROWS, 2 * H2), jnp.float32),
    )(W1, b1.reshape(1, H2),
      jnp.tile(gamma, 2).reshape(1, 2 * H2),
      jnp.tile(beta, 2).reshape(1, 2 * H2),
      jnp.block([[W2, jnp.zeros((H2, H2), jnp.float32)],
                 [jnp.zeros((H2, H2), jnp.float32), W2]]),
      jnp.tile(b2, 2).reshape(1, 2 * H2), posU, posU)


def _scatter_body(codes_hbm, dst_hbm, out_hbm,
                  idx2, upd2, ones_v, obuf, degb, acc, deg, sem, sem2):
    c = lax.axis_index("c")
    t = lax.axis_index("s")
    base = c * HALF
    iota = lax.iota(jnp.int32, 16)
    zero16 = jnp.zeros((16,), jnp.float32)
    one16 = jnp.full((16,), 1.0, jnp.float32)

    def initrow(r, carry):
        for g in range(4):
            obuf[r, pl.ds(g * 16, 16)] = zero16
        return carry
    lax.fori_loop(0, CB, initrow, 0)
    for k in range(CB // 16):
        degb[pl.ds(k * 16, 16)] = zero16
    for k in range(ROW // 16):
        ones_v[pl.ds(k * 16, 16)] = one16

    def zchunk(i, carry):
        cs = jnp.minimum(t * ZSTRIDE + i * CB, ACC_ROWS - CB)
        pltpu.sync_copy(obuf, acc.at[pl.ds(cs, CB)])
        pltpu.sync_copy(degb, deg.at[pl.ds(cs, CB)])
        return carry
    lax.fori_loop(0, ZCH, zchunk, 0)
    plsc.subcore_barrier()

    def remap(idx):
        for i in range(8):
            v = idx[pl.ds(i * 16, 16)]
            inr = jnp.logical_and(v >= base, v < base + HALF)
            dumped = BIGROW + i * 16 + iota
            idx[pl.ds(i * 16, 16)] = jnp.where(inr, v - base, dumped)

    M = 2 * CRPT

    def eoff(m):
        jj = t * CRPT + (m >> 1)
        side = m & 1
        return jj * ROW + side * (EPAD // 2), side * H2, jj

    def isactive(m):
        jj = t * CRPT + (m >> 1)
        return jnp.logical_or((m & 1) == 0, jj < RREAL)

    def issue_loads(m, k):
        e0, c0, jj = eoff(m)
        pltpu.async_copy(dst_hbm.at[pl.ds(e0, ROW)], idx2.at[k], sem)
        pltpu.async_copy(
            codes_hbm.at[pl.ds(jj * ROW, ROW), pl.ds(c0, H2)],
            upd2.at[k], sem)

    def wait_loads(m, k):
        e0, c0, jj = eoff(m)
        pltpu.make_async_copy(
            dst_hbm.at[pl.ds(e0, ROW)], idx2.at[k], sem).wait()
        pltpu.make_async_copy(
            codes_hbm.at[pl.ds(jj * ROW, ROW), pl.ds(c0, H2)],
            upd2.at[k], sem).wait()

    def issue_scats(k):
        pltpu.async_copy(upd2.at[k], acc.at[idx2.at[k]], sem2, add=True)
        pltpu.async_copy(ones_v, deg.at[idx2.at[k]], sem2, add=True)

    def wait_scats(k):
        pltpu.make_async_copy(upd2.at[k], acc.at[idx2.at[k]], sem2).wait()
        pltpu.make_async_copy(ones_v, deg.at[idx2.at[k]], sem2).wait()

    @pl.when(isactive(0))
    def _():
        issue_loads(0, 0)

    @pl.when(isactive(1))
    def _():
        issue_loads(1, 1)

    def sloop(m, carry):
        k = lax.rem(m, 3)
        act = isactive(m)

        @pl.when(act)
        def _():
            wait_loads(m, k)
            remap(idx2.at[k])

        @pl.when(jnp.logical_and(m >= 1, isactive(m - 1)))
        def _():
            wait_scats(lax.rem(m + 2, 3))

        @pl.when(act)
        def _():
            issue_scats(k)

        @pl.when(jnp.logical_and(m + 2 < M, isactive(m + 2)))
        def _():
            issue_loads(m + 2, lax.rem(m + 2, 3))
        return carry

    lax.fori_loop(0, M, sloop, 0)

    @pl.when(isactive(M - 1))
    def _():
        wait_scats(lax.rem(M - 1, 3))
    plsc.subcore_barrier()

    def dchunk(i, carry):
        cs = jnp.minimum(t * TSTRIDE + i * CB, HALF - CB)
        pltpu.sync_copy(acc.at[pl.ds(cs, CB)], obuf)
        pltpu.sync_copy(deg.at[pl.ds(cs, CB)], degb)

        def drow(r, carry2):
            dvec = plsc.load_gather(degb, [jnp.full((16,), r, jnp.int32)])
            rec = 1.0 / jnp.maximum(dvec, 1.0)
            for g in range(4):
                obuf[r, pl.ds(g * 16, 16)] = obuf[r, pl.ds(g * 16, 16)] * rec
            return carry2
        lax.fori_loop(0, CB, drow, 0)
        pltpu.sync_copy(obuf, out_hbm.at[pl.ds(base + cs, CB)])
        return carry
    lax.fori_loop(0, DCH, dchunk, 0)


@jax.jit
def _scatter_call(codes, dstv):
    f = pl.kernel(
        _scatter_body,
        out_type=jax.ShapeDtypeStruct((N, H2), jnp.float32),
        mesh=plsc.VectorSubcoreMesh(core_axis_name="c", subcore_axis_name="s"),
        scratch_types=[
            pltpu.VMEM((3, ROW), jnp.int32),
            pltpu.VMEM((3, ROW, H2), jnp.float32),
            pltpu.VMEM((ROW,), jnp.float32),
            pltpu.VMEM((CB, H2), jnp.float32),
            pltpu.VMEM((CB,), jnp.float32),
            pltpu.VMEM_SHARED((ACC_ROWS, H2), jnp.float32),
            pltpu.VMEM_SHARED((ACC_ROWS,), jnp.float32),
            pltpu.SemaphoreType.DMA,
            pltpu.SemaphoreType.DMA,
        ],
        compiler_params=_SC_PARAMS,
    )
    return f(codes, dstv)


def kernel(pos, edge_index, batch, W1, b1, gamma, beta, W2, b2):
    srcv = edge_index[0]
    dstv = edge_index[1]
    pos16 = jnp.pad(pos, ((0, 0), (0, PW - 3)))
    posU = _gather_call(pos16, srcv, dstv)
    codes = _mlp_call(W1, b1, gamma, beta, W2, b2, posU)
    return _scatter_call(codes, dstv)


# speedup vs baseline: 1.0480x; 1.0480x over previous
import jax
import jax.numpy as jnp
from jax import lax
from jax.experimental import pallas as pl
from jax.experimental.pallas import tpu as pltpu
from jax.experimental.pallas import tpu_sc as plsc

N = 50000
E = 800000
H2 = 64
ROW = 128
NROWS = 6400
EPAD = NROWS * ROW
RROWS = E // ROW
NC, NS = 2, 16
NW = NC * NS
HALF = N // NC
ACC_ROWS = 25128
BIGROW = 25000
CROWS = EPAD // 2

RPW = 196
KBA = 7
PW = 16
PSTRIPE = 3128

HROWS = NROWS // 2
CRPT = HROWS // NS
RREAL = 3050
CB = 48
ZCH = 33
DCH = 33
TSTRIDE = 1568
ZSTRIDE = 1576

_SC_PARAMS = pltpu.CompilerParams(use_tc_tiling_on_sc=False,
                                  needs_layout_passes=False)


def _rsqrt_sc(d2):
    bits = plsc.bitcast(d2, jnp.int32)
    y = plsc.bitcast(0x5F3759DF - (bits >> 1), jnp.float32)
    for _ in range(3):
        y = y * (1.5 - (0.5 * d2 * y) * y)
    return y


def _gather_body(pos_hbm, src_hbm, dst_hbm, posU_hbm,
                 sidx, didx, sbuf, dbuf, planes, pos_sh, sem, sem2):
    c = lax.axis_index("c")
    s = lax.axis_index("s")
    wid = s * NC + c
    iota = lax.iota(jnp.int32, 16)

    soff = jnp.minimum(s * PSTRIPE, N - PSTRIPE)
    pltpu.sync_copy(pos_hbm.at[pl.ds(soff, PSTRIPE)],
                    pos_sh.at[pl.ds(soff, PSTRIPE)])
    plsc.subcore_barrier()

    start = jnp.minimum(wid * RPW, RROWS - RPW)
    NB = RPW // KBA

    def issue_idx(b, k):
        r0 = start + b * KBA
        pltpu.async_copy(src_hbm.at[pl.ds(r0 * ROW, KBA * ROW)],
                         sidx.at[k], sem)
        pltpu.async_copy(dst_hbm.at[pl.ds(r0 * ROW, KBA * ROW)],
                         didx.at[k], sem)

    def wait_idx(b, k):
        r0 = start + b * KBA
        pltpu.make_async_copy(src_hbm.at[pl.ds(r0 * ROW, KBA * ROW)],
                              sidx.at[k], sem).wait()
        pltpu.make_async_copy(dst_hbm.at[pl.ds(r0 * ROW, KBA * ROW)],
                              didx.at[k], sem).wait()

    def fire_gathers(k):
        for j in range(KBA):
            pltpu.async_copy(pos_sh.at[sidx.at[k, pl.ds(j * ROW, ROW)]],
                             sbuf.at[k, pl.ds(j * ROW, ROW)], sem2)
            pltpu.async_copy(pos_sh.at[didx.at[k, pl.ds(j * ROW, ROW)]],
                             dbuf.at[k, pl.ds(j * ROW, ROW)], sem2)

    def wait_gathers(k):
        for j in range(KBA):
            pltpu.make_async_copy(
                pos_sh.at[sidx.at[k, pl.ds(j * ROW, ROW)]],
                sbuf.at[k, pl.ds(j * ROW, ROW)], sem2).wait()
            pltpu.make_async_copy(
                pos_sh.at[didx.at[k, pl.ds(j * ROW, ROW)]],
                dbuf.at[k, pl.ds(j * ROW, ROW)], sem2).wait()

    def compute(b, k):
        def grp(g, carry2):
            ridx = g * 16 + iota
            grow = 3 * (g // 8)
            lo = (g % 8) * 16
            comp = []
            for cc in range(3):
                cidx = jnp.full((16,), cc, jnp.int32)
                xs = plsc.load_gather(sbuf.at[k], [ridx, cidx])
                xd = plsc.load_gather(dbuf.at[k], [ridx, cidx])
                comp.append(xd - xs)
            d2 = comp[0] * comp[0] + comp[1] * comp[1] + comp[2] * comp[2]
            r = _rsqrt_sc(d2)
            f = 1.0 / (d2 * r + 1e-6)
            for cc in range(3):
                planes[k, grow + cc, pl.ds(lo, 16)] = comp[cc] * f
            return carry2
        lax.fori_loop(0, KBA * 8, grp, 0)
        r0 = start + b * KBA
        pltpu.sync_copy(planes.at[k], posU_hbm.at[pl.ds(3 * r0, 3 * KBA)])

    issue_idx(0, 0)
    wait_idx(0, 0)
    fire_gathers(0)
    issue_idx(1, 1)

    def bloop(b, carry):
        k = lax.rem(b, 2)
        k1 = lax.rem(b + 1, 2)

        @pl.when(b + 1 < NB)
        def _():
            wait_idx(b + 1, k1)
            fire_gathers(k1)
        wait_gathers(k)
        compute(b, k)

        @pl.when(b + 2 < NB)
        def _():
            issue_idx(b + 2, k)
        return carry

    lax.fori_loop(0, NB, bloop, 0)


@jax.jit
def _gather_call(pos16, srcv, dstv):
    f = pl.kernel(
        _gather_body,
        out_type=jax.ShapeDtypeStruct((3 * NROWS, ROW), jnp.float32),
        mesh=plsc.VectorSubcoreMesh(core_axis_name="c", subcore_axis_name="s"),
        scratch_types=[
            pltpu.VMEM((2, KBA * ROW), jnp.int32),
            pltpu.VMEM((2, KBA * ROW), jnp.int32),
            pltpu.VMEM((2, KBA * ROW, PW), jnp.float32),
            pltpu.VMEM((2, KBA * ROW, PW), jnp.float32),
            pltpu.VMEM((2, 3 * KBA, ROW), jnp.float32),
            pltpu.VMEM_SHARED((N, PW), jnp.float32),
            pltpu.SemaphoreType.DMA,
            pltpu.SemaphoreType.DMA,
        ],
        compiler_params=_SC_PARAMS,
    )
    return f(pos16, srcv, dstv)


BE = 4096


def _mlp_body(w1_ref, b1_ref, gam2_ref, bet2_ref, w2d_ref, b22_ref,
              ul_ref, ur_ref, o_ref):
    w1 = w1_ref[...]
    b1 = b1_ref[...]
    gam2 = gam2_ref[...]
    bet2 = bet2_ref[...]
    w2d = w2d_ref[...]
    b22 = b22_ref[...]

    def mean64(v):
        return jnp.mean(v, axis=1, keepdims=True)

    x0, x1, x2 = w1[0:1], w1[1:2], w1[2:3]
    z1 = jnp.zeros((1, 1), jnp.float32)
    muW = jnp.concatenate(
        [mean64(x0), mean64(x1), mean64(x2),
         z1, z1, z1, z1, z1, z1, mean64(b1)], axis=0)
    qW = jnp.concatenate(
        [2 * mean64(b1 * x0), 2 * mean64(b1 * x1), 2 * mean64(b1 * x2),
         mean64(x0 * x0), mean64(x1 * x1), mean64(x2 * x2),
         2 * mean64(x0 * x1), 2 * mean64(x0 * x2), 2 * mean64(x1 * x2),
         mean64(b1 * b1)], axis=0)
    Wmq = jnp.concatenate([jnp.broadcast_to(muW, (10, H2)),
                           jnp.broadcast_to(qW, (10, H2))], axis=1)
    W1a = jnp.concatenate([w1, jnp.zeros((6, H2), jnp.float32), b1], axis=0)
    ones128 = jnp.ones((1, ROW), jnp.float32)

    def half(u_ref):
        hs, ms = [], []
        for g in range(BE // ROW):
            ug = u_ref[3 * g:3 * g + 3, :]
            ux, uy, uz = ug[0:1], ug[1:2], ug[2:3]
            F = jnp.concatenate(
                [ug, ux * ux, uy * uy, uz * uz,
                 ux * uy, ux * uz, uy * uz, ones128], axis=0)
            hs.append(lax.dot_general(F, W1a, (((0,), (0,)), ((), ())),
                                      preferred_element_type=jnp.float32))
            ms.append(lax.dot_general(F, Wmq, (((0,), (0,)), ((), ())),
                                      preferred_element_type=jnp.float32))
        return jnp.concatenate(hs, axis=0), jnp.concatenate(ms, axis=0)

    HL, ML = half(ul_ref)
    HR, MR = half(ur_ref)
    H = jnp.concatenate([HL, HR], axis=1)
    MU = jnp.concatenate([ML[:, :H2], MR[:, :H2]], axis=1)
    Q = jnp.concatenate([ML[:, H2:], MR[:, H2:]], axis=1)
    var = Q - MU * MU
    hn = (H - MU) * lax.rsqrt(var + 1e-5) * gam2 + bet2
    g = 0.5 * hn * (1.0 + lax.erf(hn * 0.7071067811865476))
    o_ref[...] = lax.dot_general(g, w2d, (((1,), (0,)), ((), ())),
                                 preferred_element_type=jnp.float32) + b22


@jax.jit
def _mlp_call(W1, b1, gamma, beta, W2, b2, posU):
    grid = (CROWS // BE,)
    full = lambda i: (0, 0)
    return pl.pallas_call(
        _mlp_body,
        grid=grid,
        in_specs=[
            pl.BlockSpec((3, H2), full),
            pl.BlockSpec((1, H2), full),
            pl.BlockSpec((1, 2 * H2), full),
            pl.BlockSpec((1, 2 * H2), full),
            pl.BlockSpec((2 * H2, 2 * H2), full),
            pl.BlockSpec((1, 2 * H2), full),
            pl.BlockSpec((3 * BE // ROW, ROW), lambda i: (i, 0)),
            pl.BlockSpec((3 * BE // ROW, ROW),
                         lambda i: (i + CROWS // BE, 0)),
        ],
        out_specs=pl.BlockSpec((BE, 2 * H2), lambda i: (i, 0)),
        out_shape=jax.ShapeDtypeStruct((CROWS, 2 * H2), jnp.float32),
    )(W1, b1.reshape(1, H2),
      jnp.tile(gamma, 2).reshape(1, 2 * H2),
      jnp.tile(beta, 2).reshape(1, 2 * H2),
      jnp.block([[W2, jnp.zeros((H2, H2), jnp.float32)],
                 [jnp.zeros((H2, H2), jnp.float32), W2]]),
      jnp.tile(b2, 2).reshape(1, 2 * H2), posU, posU)


def _scatter_body(codes_hbm, dst_hbm, out_hbm,
                  idx2, upd2, ones_v, obuf, degb, acc, deg, sem, sem2):
    c = lax.axis_index("c")
    t = lax.axis_index("s")
    base = c * HALF
    iota = lax.iota(jnp.int32, 16)
    zero16 = jnp.zeros((16,), jnp.float32)
    one16 = jnp.full((16,), 1.0, jnp.float32)

    def initrow(r, carry):
        for g in range(4):
            obuf[r, pl.ds(g * 16, 16)] = zero16
        return carry
    lax.fori_loop(0, CB, initrow, 0)
    for k in range(CB // 16):
        degb[pl.ds(k * 16, 16)] = zero16
    for k in range(ROW // 16):
        ones_v[pl.ds(k * 16, 16)] = one16

    def zchunk(i, carry):
        cs = jnp.minimum(t * ZSTRIDE + i * CB, ACC_ROWS - CB)
        pltpu.sync_copy(obuf, acc.at[pl.ds(cs, CB)])
        pltpu.sync_copy(degb, deg.at[pl.ds(cs, CB)])
        return carry
    lax.fori_loop(0, ZCH, zchunk, 0)
    plsc.subcore_barrier()

    def remap(idx):
        for i in range(8):
            v = idx[pl.ds(i * 16, 16)]
            inr = jnp.logical_and(v >= base, v < base + HALF)
            dumped = BIGROW + i * 16 + iota
            idx[pl.ds(i * 16, 16)] = jnp.where(inr, v - base, dumped)

    M = 2 * CRPT

    def eoff(m):
        jj = t * CRPT + (m >> 1)
        side = m & 1
        return jj * ROW + side * (EPAD // 2), side * H2, jj

    def isactive(m):
        jj = t * CRPT + (m >> 1)
        return jnp.logical_or((m & 1) == 0, jj < RREAL)

    def issue_loads(m, k):
        e0, c0, jj = eoff(m)
        pltpu.async_copy(dst_hbm.at[pl.ds(e0, ROW)], idx2.at[k], sem)
        pltpu.async_copy(
            codes_hbm.at[pl.ds(jj * ROW, ROW), pl.ds(c0, H2)],
            upd2.at[k], sem)

    def wait_loads(m, k):
        e0, c0, jj = eoff(m)
        pltpu.make_async_copy(
            dst_hbm.at[pl.ds(e0, ROW)], idx2.at[k], sem).wait()
        pltpu.make_async_copy(
            codes_hbm.at[pl.ds(jj * ROW, ROW), pl.ds(c0, H2)],
            upd2.at[k], sem).wait()

    def issue_scats(k):
        pltpu.async_copy(upd2.at[k], acc.at[idx2.at[k]], sem2, add=True)
        pltpu.async_copy(ones_v, deg.at[idx2.at[k]], sem2, add=True)

    def wait_scats(k):
        pltpu.make_async_copy(upd2.at[k], acc.at[idx2.at[k]], sem2).wait()
        pltpu.make_async_copy(ones_v, deg.at[idx2.at[k]], sem2).wait()

    @pl.when(isactive(0))
    def _():
        issue_loads(0, 0)

    @pl.when(isactive(1))
    def _():
        issue_loads(1, 1)

    def sloop(m, carry):
        k = lax.rem(m, 3)
        act = isactive(m)

        @pl.when(act)
        def _():
            wait_loads(m, k)
            remap(idx2.at[k])

        @pl.when(jnp.logical_and(m >= 1, isactive(m - 1)))
        def _():
            wait_scats(lax.rem(m + 2, 3))

        @pl.when(act)
        def _():
            issue_scats(k)

        @pl.when(jnp.logical_and(m + 2 < M, isactive(m + 2)))
        def _():
            issue_loads(m + 2, lax.rem(m + 2, 3))
        return carry

    lax.fori_loop(0, M, sloop, 0)

    @pl.when(isactive(M - 1))
    def _():
        wait_scats(lax.rem(M - 1, 3))
    plsc.subcore_barrier()

    def dchunk(i, carry):
        cs = jnp.minimum(t * TSTRIDE + i * CB, HALF - CB)
        pltpu.sync_copy(acc.at[pl.ds(cs, CB)], obuf)
        pltpu.sync_copy(deg.at[pl.ds(cs, CB)], degb)

        def drow(r, carry2):
            dvec = plsc.load_gather(degb, [jnp.full((16,), r, jnp.int32)])
            rec = 1.0 / jnp.maximum(dvec, 1.0)
            for g in range(4):
                obuf[r, pl.ds(g * 16, 16)] = obuf[r, pl.ds(g * 16, 16)] * rec
            return carry2
        lax.fori_loop(0, CB, drow, 0)
        pltpu.sync_copy(obuf, out_hbm.at[pl.ds(base + cs, CB)])
        return carry
    lax.fori_loop(0, DCH, dchunk, 0)


@jax.jit
def _scatter_call(codes, dstv):
    f = pl.kernel(
        _scatter_body,
        out_type=jax.ShapeDtypeStruct((N, H2), jnp.float32),
        mesh=plsc.VectorSubcoreMesh(core_axis_name="c", subcore_axis_name="s"),
        scratch_types=[
            pltpu.VMEM((3, ROW), jnp.int32),
            pltpu.VMEM((3, ROW, H2), jnp.float32),
            pltpu.VMEM((ROW,), jnp.float32),
            pltpu.VMEM((CB, H2), jnp.float32),
            pltpu.VMEM((CB,), jnp.float32),
            pltpu.VMEM_SHARED((ACC_ROWS, H2), jnp.float32),
            pltpu.VMEM_SHARED((ACC_ROWS,), jnp.float32),
            pltpu.SemaphoreType.DMA,
            pltpu.SemaphoreType.DMA,
        ],
        compiler_params=_SC_PARAMS,
    )
    return f(codes, dstv)


def kernel(pos, edge_index, batch, W1, b1, gamma, beta, W2, b2):
    srcv = edge_index[0]
    dstv = edge_index[1]
    pos16 = jnp.pad(pos, ((0, 0), (0, PW - 3)))
    posU = _gather_call(pos16, srcv, dstv)
    codes = _mlp_call(W1, b1, gamma, beta, W2, b2, posU)
    return _scatter_call(codes, dstv)
